# Initial kernel scaffold; baseline (speedup 1.0000x reference)
import jax
import jax.numpy as jnp
from jax.experimental import pallas as pl


def kernel(node_features, edge_index, edge_type, edge_label_index, ln_gamma, ln_beta, W_in, filt, W_layer, bn1_gamma, bn1_beta, W1, b1, bn2_gamma, bn2_beta, W2, b2):
    raise NotImplementedError("write your pallas kernel here")


# speedup vs baseline: 2.4434x; 2.4434x over previous
import functools

import jax
import jax.numpy as jnp
from jax import lax
from jax.experimental import pallas as pl
from jax.experimental.pallas import tpu as pltpu
from jax.experimental.pallas import tpu_sc as plsc

_N = 10000
_D = 128
_T = 4
_E = 320000
_EL = 320000
_NC = 2
_NS = 16
_NW = _NC * _NS
_CH = 80
_F32 = jnp.float32

_NSEG = 6
_HALF = _N // _NC
_HROWS = _HALF + 48
_EC_PT = _E // _NS
_EC_CHUNKS = _EC_PT // _CH


def _sc_mesh():
    return plsc.VectorSubcoreMesh(core_axis_name="c", subcore_axis_name="s")


def _tile_chunks(sid, n_chunks, body):
    def jb(j, c0):
        c = sid + _NS * j
        @pl.when(c < n_chunks)
        def _():
            body(c)
        return c0
    lax.fori_loop(0, (n_chunks + _NS - 1) // _NS, jb, 0)


_C_PW = _E // _NW
_C_CHUNKS = _C_PW // _CH
_CNT_B = 10240


def _sc_count_body(didx6_hbm, out_hbm, acc_v, idx_v):
    cid = lax.axis_index("c")
    sid = lax.axis_index("s")
    wid = sid * _NC + cid

    z16 = jnp.zeros((16,), _F32)
    for k in range(_NSEG):
        def zb(q, cc):
            acc_v[q, pl.ds(0, 16)] = z16
            acc_v[q, pl.ds(16, 16)] = z16
            acc_v[q, pl.ds(32, 16)] = z16
            acc_v[q, pl.ds(48, 16)] = z16
            acc_v[q, pl.ds(64, 16)] = z16
            acc_v[q, pl.ds(80, 16)] = z16
            acc_v[q, pl.ds(96, 16)] = z16
            acc_v[q, pl.ds(112, 16)] = z16
            return cc
        lax.fori_loop(0, 80, zb, 0)

        base_w = k * _E + wid * _C_PW
        def cb(c, cc):
            pltpu.sync_copy(didx6_hbm.at[pl.ds(base_w + c * _CH, _CH)], idx_v)
            for j in range(_CH // 16):
                b = idx_v[pl.ds(j * 16, 16)]
                cnts, last = plsc.scan_count(b)
                plsc.addupdate_scatter(
                    acc_v,
                    [lax.shift_right_logical(b, 7),
                     lax.bitwise_and(b, 127)],
                    cnts.astype(_F32), mask=last)
            return cc
        lax.fori_loop(0, _C_CHUNKS, cb, 0)

        pltpu.sync_copy(acc_v, out_hbm.at[k * _NW + wid])


def _sc_count(didx6):
    return pl.kernel(
        _sc_count_body,
        out_type=jax.ShapeDtypeStruct((_NSEG * _NW, 80, 128), _F32),
        mesh=_sc_mesh(),
        scratch_types=[
            pltpu.VMEM((80, 128), _F32),
            pltpu.VMEM((_CH,), jnp.int32),
        ],
        compiler_params=pltpu.CompilerParams(needs_layout_passes=False),
    )(didx6)


def _tc_count_reduce_body(h_ref, o_ref):
    o_ref[...] = jnp.sum(h_ref[...], axis=0)[None]


def _tc_count_reduce(hists):
    return pl.pallas_call(
        _tc_count_reduce_body,
        grid=(_NSEG, 10),
        in_specs=[pl.BlockSpec((_NW, 8, 128), lambda k, i: (k, i, 0))],
        out_specs=pl.BlockSpec((1, 8, 128), lambda k, i: (k, i, 0)),
        out_shape=jax.ShapeDtypeStruct((_NSEG, 80, 128), _F32),
    )(hists)


def _sc_layer_body(h4_hbm, gidx_hbm, didx2_hbm, zeros_hbm, out_hbm,
                   agg_sh, gidx_v, didx_v, rows_v, zbuf_v, gsem):
    cid = lax.axis_index("c")
    sid = lax.axis_index("s")

    pltpu.sync_copy(zeros_hbm, zbuf_v)
    def zb(c):
        pltpu.sync_copy(zbuf_v, agg_sh.at[pl.ds(c * 200, 200)])
    _tile_chunks(sid, _HALF // 200, zb)
    @pl.when(sid == 0)
    def _():
        pltpu.sync_copy(zeros_hbm.at[pl.ds(0, 48)],
                        agg_sh.at[pl.ds(_HALF, 48)])
    plsc.subcore_barrier()

    base_g = sid * _EC_PT
    base_d = cid * _E + sid * _EC_PT
    def cb(c, cc):
        pltpu.sync_copy(gidx_hbm.at[pl.ds(base_g + c * _CH, _CH)], gidx_v)
        pltpu.sync_copy(didx2_hbm.at[pl.ds(base_d + c * _CH, _CH)], didx_v)
        pltpu.async_copy(h4_hbm.at[gidx_v], rows_v, gsem).wait()
        pltpu.sync_copy(rows_v, agg_sh.at[didx_v], add=True)
        return cc
    lax.fori_loop(0, _EC_CHUNKS, cb, 0)
    plsc.subcore_barrier()

    def db(c):
        pltpu.sync_copy(agg_sh.at[pl.ds(c * 200, 200)], zbuf_v)
        pltpu.sync_copy(zbuf_v, out_hbm.at[pl.ds(cid * _HALF + c * 200, 200)])
    _tile_chunks(sid, _HALF // 200, db)


def _sc_layer(h4, gidx, didx2, zeros_l):
    return pl.kernel(
        _sc_layer_body,
        out_type=jax.ShapeDtypeStruct((_N, _D), _F32),
        mesh=_sc_mesh(),
        scratch_types=[
            pltpu.VMEM_SHARED((_HROWS, _D), _F32),
            pltpu.VMEM((_CH,), jnp.int32),
            pltpu.VMEM((_CH,), jnp.int32),
            pltpu.VMEM((_CH, _D), _F32),
            pltpu.VMEM((200, _D), _F32),
            pltpu.SemaphoreType.DMA,
        ],
    )(h4, gidx, didx2, zeros_l)


_L_PW = _EL // _NW
_L_CHUNKS = _L_PW // _CH


def _sc_decode_body(p_hbm, q_hbm, srcl_hbm, dstl_hbm, r_hbm, stats_hbm,
                    p_sh, q_sh, src_v, dst_v, p_v, q_v, r_v, stats_v,
                    stage_v, psem, qsem):
    cid = lax.axis_index("c")
    sid = lax.axis_index("s")
    wid = sid * _NC + cid
    base_w = wid * _L_PW

    def stage(c):
        pltpu.sync_copy(p_hbm.at[pl.ds(c * 200, 200)], stage_v)
        pltpu.sync_copy(stage_v, p_sh.at[pl.ds(c * 200, 200)])
        pltpu.sync_copy(q_hbm.at[pl.ds(c * 200, 200)], stage_v)
        pltpu.sync_copy(stage_v, q_sh.at[pl.ds(c * 200, 200)])
    _tile_chunks(sid, _N // 200, stage)

    z16 = jnp.zeros((16,), _F32)
    for i in range(8):
        for j in range(4):
            stats_v[i, pl.ds(j * 16, 16)] = z16
    plsc.subcore_barrier()

    def cb(c, carry):
        b = base_w + c * _CH
        pltpu.sync_copy(srcl_hbm.at[pl.ds(b, _CH)], src_v)
        pltpu.sync_copy(dstl_hbm.at[pl.ds(b, _CH)], dst_v)
        dp = pltpu.async_copy(p_sh.at[src_v], p_v, psem)
        dq = pltpu.async_copy(q_sh.at[dst_v], q_v, qsem)
        dp.wait()
        dq.wait()

        def eb(i, car):
            s0, s1, t0, t1 = car
            a0 = p_v[i, pl.ds(0, 16)] + q_v[i, pl.ds(0, 16)]
            a1 = p_v[i, pl.ds(16, 16)] + q_v[i, pl.ds(16, 16)]
            r_v[i, pl.ds(0, 16)] = a0
            r_v[i, pl.ds(16, 16)] = a1
            return (s0 + a0, s1 + a1, t0 + a0 * a0, t1 + a1 * a1)

        carry = lax.fori_loop(0, _CH, eb, carry)
        pltpu.sync_copy(r_v, r_hbm.at[pl.ds(b, _CH)])
        return carry

    s0, s1, t0, t1 = lax.fori_loop(0, _L_CHUNKS, cb, (z16, z16, z16, z16))
    stats_v[0, pl.ds(0, 16)] = s0
    stats_v[0, pl.ds(16, 16)] = s1
    stats_v[0, pl.ds(32, 16)] = t0
    stats_v[0, pl.ds(48, 16)] = t1
    pltpu.sync_copy(stats_v, stats_hbm.at[wid])


def _sc_decode(p, q, srcl, dstl):
    return pl.kernel(
        _sc_decode_body,
        out_type=(jax.ShapeDtypeStruct((_EL, 32), _F32),
                  jax.ShapeDtypeStruct((_NW, 8, 64), _F32)),
        mesh=_sc_mesh(),
        scratch_types=[
            pltpu.VMEM_SHARED((_N, 32), _F32),
            pltpu.VMEM_SHARED((_N, 32), _F32),
            pltpu.VMEM((_CH,), jnp.int32),
            pltpu.VMEM((_CH,), jnp.int32),
            pltpu.VMEM((_CH, 32), _F32),
            pltpu.VMEM((_CH, 32), _F32),
            pltpu.VMEM((_CH, 32), _F32),
            pltpu.VMEM((8, 64), _F32),
            pltpu.VMEM((200, 32), _F32),
            pltpu.SemaphoreType.DMA,
            pltpu.SemaphoreType.DMA,
        ],
        compiler_params=pltpu.CompilerParams(use_tc_tiling_on_sc=False),
    )(p, q, srcl, dstl)


_BN = 1000


def _softmax_rows(f):
    m = jnp.max(f, axis=-1, keepdims=True)
    e = jnp.exp(f - m)
    return e / jnp.sum(e, axis=-1, keepdims=True)


def _tc_prologue_body(nf_ref, g_ref, b_ref, w_ref, f_ref, out_ref):
    x = nf_ref[...]
    mu = jnp.mean(x, axis=-1, keepdims=True)
    var = jnp.mean((x - mu) * (x - mu), axis=-1, keepdims=True)
    x = (x - mu) * lax.rsqrt(var + 1e-5) * g_ref[...] + b_ref[...]
    h = jnp.dot(x, w_ref[...], preferred_element_type=_F32, precision=lax.Precision.HIGHEST)
    a = _softmax_rows(f_ref[...])[0]
    out_ref[...] = a[:, None, None] * h[None, :, :]


def _tc_prologue(nf, ln_g, ln_b, w_in, filt):
    return pl.pallas_call(
        _tc_prologue_body,
        grid=(_N // _BN,),
        in_specs=[
            pl.BlockSpec((_BN, _D), lambda i: (i, 0)),
            pl.BlockSpec((1, _D), lambda i: (0, 0)),
            pl.BlockSpec((1, _D), lambda i: (0, 0)),
            pl.BlockSpec((_D, _D), lambda i: (0, 0)),
            pl.BlockSpec((2, _T), lambda i: (0, 0)),
        ],
        out_specs=pl.BlockSpec((_T, _BN, _D), lambda i: (0, i, 0)),
        out_shape=jax.ShapeDtypeStruct((_T, _N, _D), _F32),
    )(nf, ln_g, ln_b, w_in, filt)


def _tc_layer_body(l, last, agg_ref, cnt_ref, f_ref, w_ref, out_ref):
    agg = agg_ref[...]
    a_all = _softmax_rows(f_ref[...])
    ac = a_all[l]
    cnt = cnt_ref[...]
    deg = jnp.sum(ac[None, :] * cnt[:, :_T], axis=1)
    agg = agg / jnp.clip(deg, 1e-6, None)[:, None]
    h = jnp.maximum(jnp.dot(agg, w_ref[...], preferred_element_type=_F32, precision=lax.Precision.HIGHEST), 0.0)
    if last:
        out_ref[...] = h
    else:
        an = a_all[l + 1]
        out_ref[...] = an[:, None, None] * h[None, :, :]


def _tc_layer(agg, counts, filt, w_l, l, last):
    if last:
        out_spec = pl.BlockSpec((_BN, _D), lambda i: (i, 0))
        out_shape = jax.ShapeDtypeStruct((_N, _D), _F32)
    else:
        out_spec = pl.BlockSpec((_T, _BN, _D), lambda i: (0, i, 0))
        out_shape = jax.ShapeDtypeStruct((_T, _N, _D), _F32)
    return pl.pallas_call(
        functools.partial(_tc_layer_body, l, last),
        grid=(_N // _BN,),
        in_specs=[
            pl.BlockSpec((_BN, _D), lambda i: (i, 0)),
            pl.BlockSpec((_BN, 6), lambda i: (i, 0)),
            pl.BlockSpec((2, _T), lambda i: (0, 0)),
            pl.BlockSpec((_D, _D), lambda i: (0, 0)),
        ],
        out_specs=out_spec,
        out_shape=out_shape,
    )(agg, counts, filt, w_l)


def _tc_stats_body(z_ref, c_ref, s_ref, q_ref):
    i = pl.program_id(0)
    z = z_ref[...]
    cnt = c_ref[...][:, 4:6].T
    s = jnp.dot(cnt, z, preferred_element_type=_F32, precision=lax.Precision.HIGHEST)
    q = jnp.dot(cnt, z * z, preferred_element_type=_F32, precision=lax.Precision.HIGHEST)

    @pl.when(i == 0)
    def _():
        s_ref[...] = jnp.zeros_like(s_ref)
        q_ref[...] = jnp.zeros_like(q_ref)

    s_ref[...] += s
    q_ref[...] += q


def _tc_stats(z, counts):
    return pl.pallas_call(
        _tc_stats_body,
        grid=(_N // _BN,),
        in_specs=[
            pl.BlockSpec((_BN, _D), lambda i: (i, 0)),
            pl.BlockSpec((_BN, 6), lambda i: (i, 0)),
        ],
        out_specs=(pl.BlockSpec((2, _D), lambda i: (0, 0)),
                   pl.BlockSpec((2, _D), lambda i: (0, 0))),
        out_shape=(jax.ShapeDtypeStruct((2, _D), _F32),
                   jax.ShapeDtypeStruct((2, _D), _F32)),
    )(z, counts)


def _tc_pq_body(z_ref, s_ref, q_ref, g_ref, w_ref, p_ref, qo_ref):
    mu = s_ref[...] / _EL
    var = q_ref[...] / _EL - mu * mu
    sc = g_ref[...] * lax.rsqrt(var + 1e-5)
    z = z_ref[...]
    w = w_ref[...]
    wtop = w[0] * sc[0][:, None]
    wbot = w[1] * sc[1][:, None]
    p_ref[...] = jnp.dot(z, wtop, preferred_element_type=_F32, precision=lax.Precision.HIGHEST)
    qo_ref[...] = jnp.dot(z, wbot, preferred_element_type=_F32, precision=lax.Precision.HIGHEST)


def _tc_pq(z, s, q, g1, w1):
    return pl.pallas_call(
        _tc_pq_body,
        grid=(_N // _BN,),
        in_specs=[
            pl.BlockSpec((_BN, _D), lambda i: (i, 0)),
            pl.BlockSpec((2, _D), lambda i: (0, 0)),
            pl.BlockSpec((2, _D), lambda i: (0, 0)),
            pl.BlockSpec((2, _D), lambda i: (0, 0)),
            pl.BlockSpec((2, _D, 32), lambda i: (0, 0, 0)),
        ],
        out_specs=(pl.BlockSpec((_BN, 32), lambda i: (i, 0)),
                   pl.BlockSpec((_BN, 32), lambda i: (i, 0))),
        out_shape=(jax.ShapeDtypeStruct((_N, 32), _F32),
                   jax.ShapeDtypeStruct((_N, 32), _F32)),
    )(z, s, q, g1, w1)


_BE = 2000


def _tc_final_body(r_ref, st_ref, g_ref, be_ref, w_ref, bb_ref, o_ref):
    st = st_ref[...][:, 0, :]
    ssum = jnp.sum(st[:, :32], axis=0, keepdims=True)
    qsum = jnp.sum(st[:, 32:], axis=0, keepdims=True)
    mu = ssum / _EL
    var = qsum / _EL - mu * mu
    sc = g_ref[...] * lax.rsqrt(var + 1e-5)
    sh = be_ref[...] - mu * sc
    c = jnp.maximum(r_ref[...] * sc + sh, 0.0)
    o_ref[...] = jnp.dot(c, w_ref[...], preferred_element_type=_F32, precision=lax.Precision.HIGHEST) + bb_ref[...]


def _tc_final(r, stats, g2, be2, w2, b2):
    return pl.pallas_call(
        _tc_final_body,
        grid=(_EL // _BE,),
        in_specs=[
            pl.BlockSpec((_BE, 32), lambda i: (i, 0)),
            pl.BlockSpec((_NW, 8, 64), lambda i: (0, 0, 0)),
            pl.BlockSpec((1, 32), lambda i: (0, 0)),
            pl.BlockSpec((1, 32), lambda i: (0, 0)),
            pl.BlockSpec((32, 2), lambda i: (0, 0)),
            pl.BlockSpec((1, 2), lambda i: (0, 0)),
        ],
        out_specs=pl.BlockSpec((_BE, 2), lambda i: (i, 0)),
        out_shape=jax.ShapeDtypeStruct((_EL, 2), _F32),
    )(r, stats, g2, be2, w2, b2)


def kernel(node_features, edge_index, edge_type, edge_label_index,
           ln_gamma, ln_beta, W_in, filt, W_layer,
           bn1_gamma, bn1_beta, W1, b1, bn2_gamma, bn2_beta, W2, b2):
    src = edge_index[0].astype(jnp.int32)
    dst = edge_index[1].astype(jnp.int32)
    et = edge_type.astype(jnp.int32)
    srcl = edge_label_index[0].astype(jnp.int32)
    dstl = edge_label_index[1].astype(jnp.int32)

    gidx = et * _N + src
    zeros_l = jnp.zeros((200, _D), _F32)

    eidx = jnp.arange(_E, dtype=jnp.int32)
    dump_n = _N + (eidx % 128)
    didx6 = jnp.concatenate(
        [jnp.where(et == t, dst, dump_n) for t in range(_T)] + [srcl, dstl])
    counts = _tc_count_reduce(_sc_count(didx6))
    counts = counts.reshape(_NSEG, _CNT_B).T

    dump_h = _HALF + (eidx % 48)
    didx2 = jnp.concatenate([
        jnp.where((dst >= c * _HALF) & (dst < (c + 1) * _HALF),
                  dst - c * _HALF, dump_h)
        for c in range(_NC)])

    h0 = _tc_prologue(node_features, ln_gamma.reshape(1, _D),
                      ln_beta.reshape(1, _D), W_in, filt)

    agg0 = _sc_layer(h0.reshape(_T * _N, _D), gidx, didx2, zeros_l)
    h1 = _tc_layer(agg0, counts, filt, W_layer[0], l=0, last=False)
    agg1 = _sc_layer(h1.reshape(_T * _N, _D), gidx, didx2, zeros_l)
    z = _tc_layer(agg1, counts, filt, W_layer[1], l=1, last=True)

    s, q = _tc_stats(z, counts)
    p_rows, q_rows = _tc_pq(z, s, q, bn1_gamma.reshape(2, _D),
                            W1.reshape(2, _D, 32))
    r, stats = _sc_decode(p_rows, q_rows, srcl, dstl)
    return _tc_final(r, stats, bn2_gamma.reshape(1, 32),
                     bn2_beta.reshape(1, 32), W2, b2.reshape(1, 2))


# speedup vs baseline: 4.7351x; 1.9379x over previous
import functools

import jax
import jax.numpy as jnp
from jax import lax
from jax.experimental import pallas as pl
from jax.experimental.pallas import tpu as pltpu
from jax.experimental.pallas import tpu_sc as plsc

_N = 10000
_D = 128
_T = 4
_E = 320000
_EL = 320000
_NC = 2
_NS = 16
_NW = _NC * _NS
_CH = 80
_F32 = jnp.float32

_NSEG = 6
_HALF = _N // _NC
_HROWS = _HALF + 48
_EC_PT = _E // _NS
_EC_CHUNKS = _EC_PT // _CH


def _sc_mesh():
    return plsc.VectorSubcoreMesh(core_axis_name="c", subcore_axis_name="s")


def _tile_chunks(sid, n_chunks, body):
    def jb(j, c0):
        c = sid + _NS * j
        @pl.when(c < n_chunks)
        def _():
            body(c)
        return c0
    lax.fori_loop(0, (n_chunks + _NS - 1) // _NS, jb, 0)


_C_PW = _E // _NW
_C_CHUNKS = _C_PW // _CH
_CNT_B = 10240


def _sc_count_body(didx6_hbm, out_hbm, *bufs):
    accs = bufs[0:_NSEG]
    idx_v = bufs[_NSEG:_NSEG + 5]
    isem = bufs[_NSEG + 5:_NSEG + 10]
    cid = lax.axis_index("c")
    sid = lax.axis_index("s")
    wid = sid * _NC + cid

    z16 = jnp.zeros((16,), _F32)
    for k in range(_NSEG):
        def zb(q, cc):
            accs[k][lax.shift_right_logical(q, 3),
                    pl.ds(lax.bitwise_and(q, 7) * 16, 16)] = z16
            return cc
        lax.fori_loop(0, 640, zb, 0)

    base_w = wid * _C_CHUNKS * (_NSEG * _CH)

    def blk(g, cc):
        c0 = g * 5
        idesc = []
        for b in range(5):
            idesc.append(pltpu.async_copy(
                didx6_hbm.at[pl.ds(base_w + (c0 + b) * (_NSEG * _CH),
                                   _NSEG * _CH)],
                idx_v[b], isem[b]))
        for b in range(5):
            idesc[b].wait()
            for k in range(_NSEG):
                for j in range(_CH // 16):
                    v = idx_v[b][pl.ds(k * _CH + j * 16, 16)]
                    cnts, last = plsc.scan_count(v)
                    plsc.addupdate_scatter(
                        accs[k],
                        [lax.shift_right_logical(v, 7),
                         lax.bitwise_and(v, 127)],
                        cnts.astype(_F32), mask=last)
        return cc
    lax.fori_loop(0, _C_CHUNKS // 5, blk, 0)

    for k in range(_NSEG):
        pltpu.sync_copy(accs[k], out_hbm.at[k * _NW + wid])


def _sc_count(didx6):
    scratch = ([pltpu.VMEM((80, 128), _F32) for _ in range(_NSEG)]
               + [pltpu.VMEM((_NSEG * _CH,), jnp.int32) for _ in range(5)]
               + [pltpu.SemaphoreType.DMA for _ in range(5)])
    return pl.kernel(
        _sc_count_body,
        out_type=jax.ShapeDtypeStruct((_NSEG * _NW, 80, 128), _F32),
        mesh=_sc_mesh(),
        scratch_types=scratch,
        compiler_params=pltpu.CompilerParams(needs_layout_passes=False),
    )(didx6)


def _tc_count_reduce_body(h_ref, o_ref):
    o_ref[...] = jnp.sum(h_ref[...], axis=0)[None]


def _tc_count_reduce(hists):
    return pl.pallas_call(
        _tc_count_reduce_body,
        grid=(_NSEG, 10),
        in_specs=[pl.BlockSpec((_NW, 8, 128), lambda k, i: (k, i, 0))],
        out_specs=pl.BlockSpec((1, 8, 128), lambda k, i: (k, i, 0)),
        out_shape=jax.ShapeDtypeStruct((_NSEG, 80, 128), _F32),
    )(hists)


_NB = 5


def _sc_layer_body(h4_hbm, comb_hbm, zeros_hbm, out_hbm, agg_sh, *bufs):
    idx_v = bufs[0:_NB]
    gidx_v = bufs[_NB:2 * _NB]
    didx_v = bufs[2 * _NB:3 * _NB]
    rows_v = bufs[3 * _NB:4 * _NB]
    zbuf_v = bufs[4 * _NB]
    isem = bufs[4 * _NB + 1:4 * _NB + 1 + _NB]
    gsem = bufs[4 * _NB + 1 + _NB:4 * _NB + 1 + 2 * _NB]
    ssem = bufs[4 * _NB + 1 + 2 * _NB:4 * _NB + 1 + 3 * _NB]
    cid = lax.axis_index("c")
    sid = lax.axis_index("s")

    pltpu.sync_copy(zeros_hbm, zbuf_v)
    def zb(c):
        pltpu.sync_copy(zbuf_v, agg_sh.at[pl.ds(c * 200, 200)])
    _tile_chunks(sid, _HALF // 200, zb)
    @pl.when(sid == 0)
    def _():
        pltpu.sync_copy(zeros_hbm.at[pl.ds(0, 48)],
                        agg_sh.at[pl.ds(_HALF, 48)])
    plsc.subcore_barrier()

    base = (cid * _NS + sid) * _EC_CHUNKS * (2 * _CH)

    def blk(g, cc):
        c0 = g * _NB
        idesc = []
        for b in range(_NB):
            @pl.when(g > 0)
            def _(b=b):
                pltpu.make_async_copy(
                    rows_v[b], agg_sh.at[didx_v[b]], ssem[b]).wait()
            idesc.append(pltpu.async_copy(
                comb_hbm.at[pl.ds(base + (c0 + b) * (2 * _CH), 2 * _CH)],
                idx_v[b], isem[b]))
        gdesc = []
        for b in range(_NB):
            idesc[b].wait()
            for j in range(_CH // 16):
                gidx_v[b][pl.ds(j * 16, 16)] = idx_v[b][pl.ds(j * 16, 16)]
                didx_v[b][pl.ds(j * 16, 16)] = \
                    idx_v[b][pl.ds(_CH + j * 16, 16)]
            gdesc.append(pltpu.async_copy(
                h4_hbm.at[gidx_v[b]], rows_v[b], gsem[b]))
        for b in range(_NB):
            gdesc[b].wait()
            pltpu.async_copy(rows_v[b], agg_sh.at[didx_v[b]], ssem[b],
                             add=True)
        return cc
    lax.fori_loop(0, _EC_CHUNKS // _NB, blk, 0)
    for b in range(_NB):
        pltpu.make_async_copy(rows_v[b], agg_sh.at[didx_v[b]], ssem[b]).wait()
    plsc.subcore_barrier()

    def db(c):
        pltpu.sync_copy(agg_sh.at[pl.ds(c * 200, 200)], zbuf_v)
        pltpu.sync_copy(zbuf_v, out_hbm.at[pl.ds(cid * _HALF + c * 200, 200)])
    _tile_chunks(sid, _HALF // 200, db)


def _sc_layer(h4, comb, zeros_l):
    scratch = ([pltpu.VMEM_SHARED((_HROWS, _D), _F32)]
               + [pltpu.VMEM((2 * _CH,), jnp.int32) for _ in range(_NB)]
               + [pltpu.VMEM((_CH,), jnp.int32) for _ in range(2 * _NB)]
               + [pltpu.VMEM((_CH, _D), _F32) for _ in range(_NB)]
               + [pltpu.VMEM((200, _D), _F32)]
               + [pltpu.SemaphoreType.DMA for _ in range(3 * _NB)])
    return pl.kernel(
        _sc_layer_body,
        out_type=jax.ShapeDtypeStruct((_N, _D), _F32),
        mesh=_sc_mesh(),
        scratch_types=scratch,
    )(h4, comb, zeros_l)


_L_PW = _EL // _NW
_L_CHUNKS = _L_PW // _CH


def _sc_decode_body(p_hbm, q_hbm, lab_hbm, r_hbm, stats_hbm,
                    p_sh, q_sh, stats_v, stage_v, *bufs):
    idx_v = bufs[0:_NB]
    src_v = bufs[_NB:2 * _NB]
    dst_v = bufs[2 * _NB:3 * _NB]
    p_v = bufs[3 * _NB:4 * _NB]
    q_v = bufs[4 * _NB:5 * _NB]
    r_v = bufs[5 * _NB:6 * _NB]
    isem = bufs[6 * _NB:7 * _NB]
    psem = bufs[7 * _NB:8 * _NB]
    qsem = bufs[8 * _NB:9 * _NB]
    wsem = bufs[9 * _NB:10 * _NB]
    cid = lax.axis_index("c")
    sid = lax.axis_index("s")
    wid = sid * _NC + cid
    base_w = wid * _L_PW

    def stage(c):
        pltpu.sync_copy(p_hbm.at[pl.ds(c * 200, 200)], stage_v)
        pltpu.sync_copy(stage_v, p_sh.at[pl.ds(c * 200, 200)])
        pltpu.sync_copy(q_hbm.at[pl.ds(c * 200, 200)], stage_v)
        pltpu.sync_copy(stage_v, q_sh.at[pl.ds(c * 200, 200)])
    _tile_chunks(sid, _N // 200, stage)

    z16 = jnp.zeros((16,), _F32)
    for i in range(8):
        for j in range(4):
            stats_v[i, pl.ds(j * 16, 16)] = z16
    plsc.subcore_barrier()

    base_i = wid * _L_CHUNKS * (2 * _CH)

    def blk(g, carry):
        c0 = g * _NB
        idesc = []
        for b in range(_NB):
            @pl.when(g > 0)
            def _(b=b):
                pltpu.make_async_copy(
                    r_v[b], r_hbm.at[pl.ds(base_w, _CH)], wsem[b]).wait()
            idesc.append(pltpu.async_copy(
                lab_hbm.at[pl.ds(base_i + (c0 + b) * (2 * _CH), 2 * _CH)],
                idx_v[b], isem[b]))
        gdesc = []
        for b in range(_NB):
            idesc[b].wait()
            for j in range(_CH // 16):
                src_v[b][pl.ds(j * 16, 16)] = idx_v[b][pl.ds(j * 16, 16)]
                dst_v[b][pl.ds(j * 16, 16)] = \
                    idx_v[b][pl.ds(_CH + j * 16, 16)]
            gdesc.append((pltpu.async_copy(p_sh.at[src_v[b]], p_v[b], psem[b]),
                          pltpu.async_copy(q_sh.at[dst_v[b]], q_v[b], qsem[b])))
        for b in range(_NB):
            gdesc[b][0].wait()
            gdesc[b][1].wait()

            def eb(i, car):
                s0, s1, t0, t1 = car
                a0 = p_v[b][i, pl.ds(0, 16)] + q_v[b][i, pl.ds(0, 16)]
                a1 = p_v[b][i, pl.ds(16, 16)] + q_v[b][i, pl.ds(16, 16)]
                r_v[b][i, pl.ds(0, 16)] = a0
                r_v[b][i, pl.ds(16, 16)] = a1
                return (s0 + a0, s1 + a1, t0 + a0 * a0, t1 + a1 * a1)

            carry = lax.fori_loop(0, _CH, eb, carry)
            pltpu.async_copy(
                r_v[b], r_hbm.at[pl.ds(base_w + (c0 + b) * _CH, _CH)],
                wsem[b])
        return carry

    s0, s1, t0, t1 = lax.fori_loop(0, _L_CHUNKS // _NB, blk,
                                   (z16, z16, z16, z16))
    for b in range(_NB):
        pltpu.make_async_copy(
            r_v[b], r_hbm.at[pl.ds(base_w, _CH)], wsem[b]).wait()
    stats_v[0, pl.ds(0, 16)] = s0
    stats_v[0, pl.ds(16, 16)] = s1
    stats_v[0, pl.ds(32, 16)] = t0
    stats_v[0, pl.ds(48, 16)] = t1
    pltpu.sync_copy(stats_v, stats_hbm.at[wid])


def _sc_decode(p, q, lab):
    scratch = ([pltpu.VMEM_SHARED((_N, 32), _F32),
                pltpu.VMEM_SHARED((_N, 32), _F32),
                pltpu.VMEM((8, 64), _F32),
                pltpu.VMEM((200, 32), _F32)]
               + [pltpu.VMEM((2 * _CH,), jnp.int32) for _ in range(_NB)]
               + [pltpu.VMEM((_CH,), jnp.int32) for _ in range(2 * _NB)]
               + [pltpu.VMEM((_CH, 32), _F32) for _ in range(3 * _NB)]
               + [pltpu.SemaphoreType.DMA for _ in range(4 * _NB)])
    return pl.kernel(
        _sc_decode_body,
        out_type=(jax.ShapeDtypeStruct((_EL, 32), _F32),
                  jax.ShapeDtypeStruct((_NW, 8, 64), _F32)),
        mesh=_sc_mesh(),
        scratch_types=scratch,
        compiler_params=pltpu.CompilerParams(use_tc_tiling_on_sc=False),
    )(p, q, lab)


_BN = 1000


def _softmax_rows(f):
    m = jnp.max(f, axis=-1, keepdims=True)
    e = jnp.exp(f - m)
    return e / jnp.sum(e, axis=-1, keepdims=True)


def _tc_prologue_body(nf_ref, g_ref, b_ref, w_ref, f_ref, out_ref):
    x = nf_ref[...]
    mu = jnp.mean(x, axis=-1, keepdims=True)
    var = jnp.mean((x - mu) * (x - mu), axis=-1, keepdims=True)
    x = (x - mu) * lax.rsqrt(var + 1e-5) * g_ref[...] + b_ref[...]
    h = jnp.dot(x, w_ref[...], preferred_element_type=_F32, precision=lax.Precision.HIGHEST)
    a = _softmax_rows(f_ref[...])[0]
    out_ref[...] = a[:, None, None] * h[None, :, :]


def _tc_prologue(nf, ln_g, ln_b, w_in, filt):
    return pl.pallas_call(
        _tc_prologue_body,
        grid=(_N // _BN,),
        in_specs=[
            pl.BlockSpec((_BN, _D), lambda i: (i, 0)),
            pl.BlockSpec((1, _D), lambda i: (0, 0)),
            pl.BlockSpec((1, _D), lambda i: (0, 0)),
            pl.BlockSpec((_D, _D), lambda i: (0, 0)),
            pl.BlockSpec((2, _T), lambda i: (0, 0)),
        ],
        out_specs=pl.BlockSpec((_T, _BN, _D), lambda i: (0, i, 0)),
        out_shape=jax.ShapeDtypeStruct((_T, _N, _D), _F32),
    )(nf, ln_g, ln_b, w_in, filt)


def _tc_layer_body(l, last, agg_ref, cnt_ref, f_ref, w_ref, out_ref):
    agg = agg_ref[...]
    a_all = _softmax_rows(f_ref[...])
    ac = a_all[l]
    cnt = cnt_ref[...]
    deg = jnp.sum(ac[None, :] * cnt[:, :_T], axis=1)
    agg = agg / jnp.clip(deg, 1e-6, None)[:, None]
    h = jnp.maximum(jnp.dot(agg, w_ref[...], preferred_element_type=_F32, precision=lax.Precision.HIGHEST), 0.0)
    if last:
        out_ref[...] = h
    else:
        an = a_all[l + 1]
        out_ref[...] = an[:, None, None] * h[None, :, :]


def _tc_layer(agg, counts, filt, w_l, l, last):
    if last:
        out_spec = pl.BlockSpec((_BN, _D), lambda i: (i, 0))
        out_shape = jax.ShapeDtypeStruct((_N, _D), _F32)
    else:
        out_spec = pl.BlockSpec((_T, _BN, _D), lambda i: (0, i, 0))
        out_shape = jax.ShapeDtypeStruct((_T, _N, _D), _F32)
    return pl.pallas_call(
        functools.partial(_tc_layer_body, l, last),
        grid=(_N // _BN,),
        in_specs=[
            pl.BlockSpec((_BN, _D), lambda i: (i, 0)),
            pl.BlockSpec((_BN, 6), lambda i: (i, 0)),
            pl.BlockSpec((2, _T), lambda i: (0, 0)),
            pl.BlockSpec((_D, _D), lambda i: (0, 0)),
        ],
        out_specs=out_spec,
        out_shape=out_shape,
    )(agg, counts, filt, w_l)


def _tc_stats_body(z_ref, c_ref, s_ref, q_ref):
    i = pl.program_id(0)
    z = z_ref[...]
    cnt = c_ref[...][:, 4:6].T
    s = jnp.dot(cnt, z, preferred_element_type=_F32, precision=lax.Precision.HIGHEST)
    q = jnp.dot(cnt, z * z, preferred_element_type=_F32, precision=lax.Precision.HIGHEST)

    @pl.when(i == 0)
    def _():
        s_ref[...] = jnp.zeros_like(s_ref)
        q_ref[...] = jnp.zeros_like(q_ref)

    s_ref[...] += s
    q_ref[...] += q


def _tc_stats(z, counts):
    return pl.pallas_call(
        _tc_stats_body,
        grid=(_N // _BN,),
        in_specs=[
            pl.BlockSpec((_BN, _D), lambda i: (i, 0)),
            pl.BlockSpec((_BN, 6), lambda i: (i, 0)),
        ],
        out_specs=(pl.BlockSpec((2, _D), lambda i: (0, 0)),
                   pl.BlockSpec((2, _D), lambda i: (0, 0))),
        out_shape=(jax.ShapeDtypeStruct((2, _D), _F32),
                   jax.ShapeDtypeStruct((2, _D), _F32)),
    )(z, counts)


def _tc_pq_body(z_ref, s_ref, q_ref, g_ref, w_ref, p_ref, qo_ref):
    mu = s_ref[...] / _EL
    var = q_ref[...] / _EL - mu * mu
    sc = g_ref[...] * lax.rsqrt(var + 1e-5)
    z = z_ref[...]
    w = w_ref[...]
    wtop = w[0] * sc[0][:, None]
    wbot = w[1] * sc[1][:, None]
    p_ref[...] = jnp.dot(z, wtop, preferred_element_type=_F32, precision=lax.Precision.HIGHEST)
    qo_ref[...] = jnp.dot(z, wbot, preferred_element_type=_F32, precision=lax.Precision.HIGHEST)


def _tc_pq(z, s, q, g1, w1):
    return pl.pallas_call(
        _tc_pq_body,
        grid=(_N // _BN,),
        in_specs=[
            pl.BlockSpec((_BN, _D), lambda i: (i, 0)),
            pl.BlockSpec((2, _D), lambda i: (0, 0)),
            pl.BlockSpec((2, _D), lambda i: (0, 0)),
            pl.BlockSpec((2, _D), lambda i: (0, 0)),
            pl.BlockSpec((2, _D, 32), lambda i: (0, 0, 0)),
        ],
        out_specs=(pl.BlockSpec((_BN, 32), lambda i: (i, 0)),
                   pl.BlockSpec((_BN, 32), lambda i: (i, 0))),
        out_shape=(jax.ShapeDtypeStruct((_N, 32), _F32),
                   jax.ShapeDtypeStruct((_N, 32), _F32)),
    )(z, s, q, g1, w1)


_BE = 2000


def _tc_final_body(r_ref, st_ref, g_ref, be_ref, w_ref, bb_ref, o_ref):
    st = st_ref[...][:, 0, :]
    ssum = jnp.sum(st[:, :32], axis=0, keepdims=True)
    qsum = jnp.sum(st[:, 32:], axis=0, keepdims=True)
    mu = ssum / _EL
    var = qsum / _EL - mu * mu
    sc = g_ref[...] * lax.rsqrt(var + 1e-5)
    sh = be_ref[...] - mu * sc
    c = jnp.maximum(r_ref[...] * sc + sh, 0.0)
    o_ref[...] = jnp.dot(c, w_ref[...], preferred_element_type=_F32, precision=lax.Precision.HIGHEST) + bb_ref[...]


def _tc_final(r, stats, g2, be2, w2, b2):
    return pl.pallas_call(
        _tc_final_body,
        grid=(_EL // _BE,),
        in_specs=[
            pl.BlockSpec((_BE, 32), lambda i: (i, 0)),
            pl.BlockSpec((_NW, 8, 64), lambda i: (0, 0, 0)),
            pl.BlockSpec((1, 32), lambda i: (0, 0)),
            pl.BlockSpec((1, 32), lambda i: (0, 0)),
            pl.BlockSpec((32, 2), lambda i: (0, 0)),
            pl.BlockSpec((1, 2), lambda i: (0, 0)),
        ],
        out_specs=pl.BlockSpec((_BE, 2), lambda i: (i, 0)),
        out_shape=jax.ShapeDtypeStruct((_EL, 2), _F32),
    )(r, stats, g2, be2, w2, b2)


def kernel(node_features, edge_index, edge_type, edge_label_index,
           ln_gamma, ln_beta, W_in, filt, W_layer,
           bn1_gamma, bn1_beta, W1, b1, bn2_gamma, bn2_beta, W2, b2):
    src = edge_index[0].astype(jnp.int32)
    dst = edge_index[1].astype(jnp.int32)
    et = edge_type.astype(jnp.int32)
    srcl = edge_label_index[0].astype(jnp.int32)
    dstl = edge_label_index[1].astype(jnp.int32)

    gidx = et * _N + src
    zeros_l = jnp.zeros((200, _D), _F32)

    eidx = jnp.arange(_E, dtype=jnp.int32)
    dump_n = _N + (eidx % 128)
    didx6 = jnp.stack(
        [jnp.where(et == t, dst, dump_n) for t in range(_T)] + [srcl, dstl])
    didx6 = jnp.transpose(didx6.reshape(_NSEG, _NW, _C_CHUNKS, _CH),
                          (1, 2, 0, 3)).reshape(-1)
    counts = _tc_count_reduce(_sc_count(didx6))
    counts = counts.reshape(_NSEG, _CNT_B).T

    dump_h = _HALF + (eidx % 48)
    didx2 = jnp.concatenate([
        jnp.where((dst >= c * _HALF) & (dst < (c + 1) * _HALF),
                  dst - c * _HALF, dump_h)
        for c in range(_NC)])
    g3 = jnp.broadcast_to(gidx.reshape(1, _NS, _EC_CHUNKS, _CH),
                          (_NC, _NS, _EC_CHUNKS, _CH))
    d3 = didx2.reshape(_NC, _NS, _EC_CHUNKS, _CH)
    comb = jnp.stack([g3, d3], axis=3).reshape(-1)
    lab = jnp.stack([srcl.reshape(_NW, _L_CHUNKS, _CH),
                     dstl.reshape(_NW, _L_CHUNKS, _CH)], axis=2).reshape(-1)

    h0 = _tc_prologue(node_features, ln_gamma.reshape(1, _D),
                      ln_beta.reshape(1, _D), W_in, filt)

    agg0 = _sc_layer(h0.reshape(_T * _N, _D), comb, zeros_l)
    h1 = _tc_layer(agg0, counts, filt, W_layer[0], l=0, last=False)
    agg1 = _sc_layer(h1.reshape(_T * _N, _D), comb, zeros_l)
    z = _tc_layer(agg1, counts, filt, W_layer[1], l=1, last=True)

    s, q = _tc_stats(z, counts)
    p_rows, q_rows = _tc_pq(z, s, q, bn1_gamma.reshape(2, _D),
                            W1.reshape(2, _D, 32))
    r, stats = _sc_decode(p_rows, q_rows, lab)
    return _tc_final(r, stats, bn2_gamma.reshape(1, 32),
                     bn2_beta.reshape(1, 32), W2, b2.reshape(1, 2))


# speedup vs baseline: 4.7514x; 1.0034x over previous
import functools

import jax
import jax.numpy as jnp
from jax import lax
from jax.experimental import pallas as pl
from jax.experimental.pallas import tpu as pltpu
from jax.experimental.pallas import tpu_sc as plsc

_N = 10000
_D = 128
_T = 4
_E = 320000
_EL = 320000
_NC = 2
_NS = 16
_NW = _NC * _NS
_CH = 80
_F32 = jnp.float32

_NSEG = 6
_HALF = _N // _NC
_HROWS = _HALF + 48
_EC_PT = _E // _NS
_EC_CHUNKS = _EC_PT // _CH


def _sc_mesh():
    return plsc.VectorSubcoreMesh(core_axis_name="c", subcore_axis_name="s")


def _tile_chunks(sid, n_chunks, body):
    def jb(j, c0):
        c = sid + _NS * j
        @pl.when(c < n_chunks)
        def _():
            body(c)
        return c0
    lax.fori_loop(0, (n_chunks + _NS - 1) // _NS, jb, 0)


_C_PW = _E // _NW
_C_CHUNKS = _C_PW // _CH
_CNT_B = 10240


def _sc_count_body(didx6_hbm, out_hbm, *bufs):
    accs = bufs[0:_NSEG]
    idx_v = bufs[_NSEG:_NSEG + 5]
    isem = bufs[_NSEG + 5:_NSEG + 10]
    cid = lax.axis_index("c")
    sid = lax.axis_index("s")
    wid = sid * _NC + cid

    z16 = jnp.zeros((16,), _F32)
    for k in range(_NSEG):
        def zb(q, cc):
            accs[k][lax.shift_right_logical(q, 3),
                    pl.ds(lax.bitwise_and(q, 7) * 16, 16)] = z16
            return cc
        lax.fori_loop(0, 640, zb, 0)

    base_w = wid * _C_CHUNKS * (_NSEG * _CH)

    def blk(g, cc):
        c0 = g * 5
        idesc = []
        for b in range(5):
            idesc.append(pltpu.async_copy(
                didx6_hbm.at[pl.ds(base_w + (c0 + b) * (_NSEG * _CH),
                                   _NSEG * _CH)],
                idx_v[b], isem[b]))
        for b in range(5):
            idesc[b].wait()
            for k in range(_NSEG):
                for j in range(_CH // 16):
                    v = idx_v[b][pl.ds(k * _CH + j * 16, 16)]
                    cnts, last = plsc.scan_count(v)
                    plsc.addupdate_scatter(
                        accs[k],
                        [lax.shift_right_logical(v, 7),
                         lax.bitwise_and(v, 127)],
                        cnts.astype(_F32), mask=last)
        return cc
    lax.fori_loop(0, _C_CHUNKS // 5, blk, 0)

    for k in range(_NSEG):
        pltpu.sync_copy(accs[k], out_hbm.at[k * _NW + wid])


def _sc_count(didx6):
    scratch = ([pltpu.VMEM((80, 128), _F32) for _ in range(_NSEG)]
               + [pltpu.VMEM((_NSEG * _CH,), jnp.int32) for _ in range(5)]
               + [pltpu.SemaphoreType.DMA for _ in range(5)])
    return pl.kernel(
        _sc_count_body,
        out_type=jax.ShapeDtypeStruct((_NSEG * _NW, 80, 128), _F32),
        mesh=_sc_mesh(),
        scratch_types=scratch,
        compiler_params=pltpu.CompilerParams(needs_layout_passes=False),
    )(didx6)


def _tc_count_reduce_body(h_ref, o_ref):
    o_ref[...] = jnp.sum(h_ref[...], axis=0)[None]


def _tc_count_reduce(hists):
    return pl.pallas_call(
        _tc_count_reduce_body,
        grid=(_NSEG, 10),
        in_specs=[pl.BlockSpec((_NW, 8, 128), lambda k, i: (k, i, 0))],
        out_specs=pl.BlockSpec((1, 8, 128), lambda k, i: (k, i, 0)),
        out_shape=jax.ShapeDtypeStruct((_NSEG, 80, 128), _F32),
    )(hists)


_NB = 5


def _sc_layer_body(h4_hbm, comb_hbm, zeros_hbm, out_hbm, agg_sh, *bufs):
    idx_v = bufs[0:_NB]
    gidx_v = bufs[_NB:2 * _NB]
    didx_v = bufs[2 * _NB:3 * _NB]
    rows_v = bufs[3 * _NB:4 * _NB]
    zbuf_v = rows_v[0]
    isem = bufs[4 * _NB:4 * _NB + _NB]
    gsem = bufs[4 * _NB + _NB:4 * _NB + 2 * _NB]
    ssem = bufs[4 * _NB + 2 * _NB:4 * _NB + 3 * _NB]
    cid = lax.axis_index("c")
    sid = lax.axis_index("s")

    pltpu.sync_copy(zeros_hbm, zbuf_v)
    def zb(c):
        pltpu.sync_copy(zbuf_v, agg_sh.at[pl.ds(c * _CH, _CH)])
    _tile_chunks(sid, 62, zb)
    @pl.when(sid == 0)
    def _():
        pltpu.sync_copy(zeros_hbm.at[pl.ds(0, 40)],
                        agg_sh.at[pl.ds(4960, 40)])
        pltpu.sync_copy(zeros_hbm.at[pl.ds(0, 48)],
                        agg_sh.at[pl.ds(_HALF, 48)])
    plsc.subcore_barrier()

    base = (cid * _NS + sid) * _EC_CHUNKS * (2 * _CH)

    def blk(g, cc):
        c0 = g * _NB
        idesc = []
        for b in range(_NB):
            @pl.when(g > 0)
            def _(b=b):
                pltpu.make_async_copy(
                    rows_v[b], agg_sh.at[didx_v[b]], ssem[b]).wait()
            idesc.append(pltpu.async_copy(
                comb_hbm.at[pl.ds(base + (c0 + b) * (2 * _CH), 2 * _CH)],
                idx_v[b], isem[b]))
        gdesc = []
        for b in range(_NB):
            idesc[b].wait()
            for j in range(_CH // 16):
                gidx_v[b][pl.ds(j * 16, 16)] = idx_v[b][pl.ds(j * 16, 16)]
                didx_v[b][pl.ds(j * 16, 16)] = \
                    idx_v[b][pl.ds(_CH + j * 16, 16)]
            gdesc.append(pltpu.async_copy(
                h4_hbm.at[gidx_v[b]], rows_v[b], gsem[b]))
        for b in range(_NB):
            gdesc[b].wait()
            pltpu.async_copy(rows_v[b], agg_sh.at[didx_v[b]], ssem[b],
                             add=True)
        return cc
    lax.fori_loop(0, _EC_CHUNKS // _NB, blk, 0)
    for b in range(_NB):
        pltpu.make_async_copy(rows_v[b], agg_sh.at[didx_v[b]], ssem[b]).wait()
    plsc.subcore_barrier()

    def db(c):
        pltpu.sync_copy(agg_sh.at[pl.ds(c * _CH, _CH)], zbuf_v)
        pltpu.sync_copy(zbuf_v, out_hbm.at[pl.ds(cid * _HALF + c * _CH, _CH)])
    _tile_chunks(sid, 62, db)
    @pl.when(sid == 0)
    def _():
        pltpu.sync_copy(agg_sh.at[pl.ds(4960, 40)],
                        zbuf_v.at[pl.ds(0, 40)])
        pltpu.sync_copy(zbuf_v.at[pl.ds(0, 40)],
                        out_hbm.at[pl.ds(cid * _HALF + 4960, 40)])


def _sc_layer(h4, comb, zeros_l):
    scratch = ([pltpu.VMEM_SHARED((_HROWS, _D), _F32)]
               + [pltpu.VMEM((2 * _CH,), jnp.int32) for _ in range(_NB)]
               + [pltpu.VMEM((_CH,), jnp.int32) for _ in range(2 * _NB)]
               + [pltpu.VMEM((_CH, _D), _F32) for _ in range(_NB)]
               + [pltpu.SemaphoreType.DMA for _ in range(3 * _NB)])
    return pl.kernel(
        _sc_layer_body,
        out_type=jax.ShapeDtypeStruct((_N, _D), _F32),
        mesh=_sc_mesh(),
        scratch_types=scratch,
    )(h4, comb, zeros_l)


_L_PW = _EL // _NW
_NBD = 5
_L_CHUNKS = _L_PW // _CH


def _sc_decode_body(p_hbm, q_hbm, lab_hbm, r_hbm, stats_hbm,
                    p_sh, q_sh, stats_v, stage_v, *bufs):
    idx_v = bufs[0:_NBD]
    src_v = bufs[_NBD:2 * _NBD]
    dst_v = bufs[2 * _NBD:3 * _NBD]
    p_v = bufs[3 * _NBD:4 * _NBD]
    q_v = bufs[4 * _NBD:5 * _NBD]
    r_v = bufs[5 * _NBD:6 * _NBD]
    isem = bufs[6 * _NBD:7 * _NBD]
    psem = bufs[7 * _NBD:8 * _NBD]
    qsem = bufs[8 * _NBD:9 * _NBD]
    wsem = bufs[9 * _NBD:10 * _NBD]
    cid = lax.axis_index("c")
    sid = lax.axis_index("s")
    wid = sid * _NC + cid
    base_w = wid * _L_PW

    def stage(c):
        pltpu.sync_copy(p_hbm.at[pl.ds(c * 200, 200)], stage_v)
        pltpu.sync_copy(stage_v, p_sh.at[pl.ds(c * 200, 200)])
        pltpu.sync_copy(q_hbm.at[pl.ds(c * 200, 200)], stage_v)
        pltpu.sync_copy(stage_v, q_sh.at[pl.ds(c * 200, 200)])
    _tile_chunks(sid, _N // 200, stage)

    z16 = jnp.zeros((16,), _F32)
    for i in range(8):
        for j in range(4):
            stats_v[i, pl.ds(j * 16, 16)] = z16
    plsc.subcore_barrier()

    base_i = wid * _L_CHUNKS * (2 * _CH)

    def blk(g, carry):
        c0 = g * _NBD
        idesc = []
        for b in range(_NBD):
            @pl.when(g > 0)
            def _(b=b):
                pltpu.make_async_copy(
                    r_v[b], r_hbm.at[pl.ds(base_w, _CH)], wsem[b]).wait()
            idesc.append(pltpu.async_copy(
                lab_hbm.at[pl.ds(base_i + (c0 + b) * (2 * _CH), 2 * _CH)],
                idx_v[b], isem[b]))
        gdesc = []
        for b in range(_NBD):
            idesc[b].wait()
            for j in range(_CH // 16):
                src_v[b][pl.ds(j * 16, 16)] = idx_v[b][pl.ds(j * 16, 16)]
                dst_v[b][pl.ds(j * 16, 16)] = \
                    idx_v[b][pl.ds(_CH + j * 16, 16)]
            gdesc.append((pltpu.async_copy(p_sh.at[src_v[b]], p_v[b], psem[b]),
                          pltpu.async_copy(q_sh.at[dst_v[b]], q_v[b], qsem[b])))
        for b in range(_NBD):
            gdesc[b][0].wait()
            gdesc[b][1].wait()

            def eb(i, car):
                s0, s1, t0, t1 = car
                a0 = p_v[b][i, pl.ds(0, 16)] + q_v[b][i, pl.ds(0, 16)]
                a1 = p_v[b][i, pl.ds(16, 16)] + q_v[b][i, pl.ds(16, 16)]
                r_v[b][i, pl.ds(0, 16)] = a0
                r_v[b][i, pl.ds(16, 16)] = a1
                return (s0 + a0, s1 + a1, t0 + a0 * a0, t1 + a1 * a1)

            carry = lax.fori_loop(0, _CH, eb, carry)
            pltpu.async_copy(
                r_v[b], r_hbm.at[pl.ds(base_w + (c0 + b) * _CH, _CH)],
                wsem[b])
        return carry

    s0, s1, t0, t1 = lax.fori_loop(0, _L_CHUNKS // _NBD, blk,
                                   (z16, z16, z16, z16))
    for b in range(_NBD):
        pltpu.make_async_copy(
            r_v[b], r_hbm.at[pl.ds(base_w, _CH)], wsem[b]).wait()
    stats_v[0, pl.ds(0, 16)] = s0
    stats_v[0, pl.ds(16, 16)] = s1
    stats_v[0, pl.ds(32, 16)] = t0
    stats_v[0, pl.ds(48, 16)] = t1
    pltpu.sync_copy(stats_v, stats_hbm.at[wid])


def _sc_decode(p, q, lab):
    scratch = ([pltpu.VMEM_SHARED((_N, 32), _F32),
                pltpu.VMEM_SHARED((_N, 32), _F32),
                pltpu.VMEM((8, 64), _F32),
                pltpu.VMEM((200, 32), _F32)]
               + [pltpu.VMEM((2 * _CH,), jnp.int32) for _ in range(_NBD)]
               + [pltpu.VMEM((_CH,), jnp.int32) for _ in range(2 * _NBD)]
               + [pltpu.VMEM((_CH, 32), _F32) for _ in range(3 * _NBD)]
               + [pltpu.SemaphoreType.DMA for _ in range(4 * _NBD)])
    return pl.kernel(
        _sc_decode_body,
        out_type=(jax.ShapeDtypeStruct((_EL, 32), _F32),
                  jax.ShapeDtypeStruct((_NW, 8, 64), _F32)),
        mesh=_sc_mesh(),
        scratch_types=scratch,
        compiler_params=pltpu.CompilerParams(use_tc_tiling_on_sc=False),
    )(p, q, lab)


_BN = 1000


def _softmax_rows(f):
    m = jnp.max(f, axis=-1, keepdims=True)
    e = jnp.exp(f - m)
    return e / jnp.sum(e, axis=-1, keepdims=True)


def _tc_prologue_body(nf_ref, g_ref, b_ref, w_ref, f_ref, out_ref):
    x = nf_ref[...]
    mu = jnp.mean(x, axis=-1, keepdims=True)
    var = jnp.mean((x - mu) * (x - mu), axis=-1, keepdims=True)
    x = (x - mu) * lax.rsqrt(var + 1e-5) * g_ref[...] + b_ref[...]
    h = jnp.dot(x, w_ref[...], preferred_element_type=_F32, precision=lax.Precision.HIGHEST)
    a = _softmax_rows(f_ref[...])[0]
    out_ref[...] = a[:, None, None] * h[None, :, :]


def _tc_prologue(nf, ln_g, ln_b, w_in, filt):
    return pl.pallas_call(
        _tc_prologue_body,
        grid=(_N // _BN,),
        in_specs=[
            pl.BlockSpec((_BN, _D), lambda i: (i, 0)),
            pl.BlockSpec((1, _D), lambda i: (0, 0)),
            pl.BlockSpec((1, _D), lambda i: (0, 0)),
            pl.BlockSpec((_D, _D), lambda i: (0, 0)),
            pl.BlockSpec((2, _T), lambda i: (0, 0)),
        ],
        out_specs=pl.BlockSpec((_T, _BN, _D), lambda i: (0, i, 0)),
        out_shape=jax.ShapeDtypeStruct((_T, _N, _D), _F32),
    )(nf, ln_g, ln_b, w_in, filt)


def _tc_layer_body(l, last, agg_ref, cnt_ref, f_ref, w_ref, out_ref):
    agg = agg_ref[...]
    a_all = _softmax_rows(f_ref[...])
    ac = a_all[l]
    cnt = cnt_ref[...]
    deg = jnp.sum(ac[None, :] * cnt[:, :_T], axis=1)
    agg = agg / jnp.clip(deg, 1e-6, None)[:, None]
    h = jnp.maximum(jnp.dot(agg, w_ref[...], preferred_element_type=_F32, precision=lax.Precision.HIGHEST), 0.0)
    if last:
        out_ref[...] = h
    else:
        an = a_all[l + 1]
        out_ref[...] = an[:, None, None] * h[None, :, :]


def _tc_layer(agg, counts, filt, w_l, l, last):
    if last:
        out_spec = pl.BlockSpec((_BN, _D), lambda i: (i, 0))
        out_shape = jax.ShapeDtypeStruct((_N, _D), _F32)
    else:
        out_spec = pl.BlockSpec((_T, _BN, _D), lambda i: (0, i, 0))
        out_shape = jax.ShapeDtypeStruct((_T, _N, _D), _F32)
    return pl.pallas_call(
        functools.partial(_tc_layer_body, l, last),
        grid=(_N // _BN,),
        in_specs=[
            pl.BlockSpec((_BN, _D), lambda i: (i, 0)),
            pl.BlockSpec((_BN, 6), lambda i: (i, 0)),
            pl.BlockSpec((2, _T), lambda i: (0, 0)),
            pl.BlockSpec((_D, _D), lambda i: (0, 0)),
        ],
        out_specs=out_spec,
        out_shape=out_shape,
    )(agg, counts, filt, w_l)


def _tc_pq_body(z_ref, c_ref, g_ref, w_ref, p_ref, qo_ref, s_acc, q_acc):
    ph = pl.program_id(0)
    i = pl.program_id(1)
    z = z_ref[...]

    @pl.when(ph == 0)
    def _():
        cnt = c_ref[...][:, 4:6].T
        s = jnp.dot(cnt, z, preferred_element_type=_F32,
                    precision=lax.Precision.HIGHEST)
        q = jnp.dot(cnt, z * z, preferred_element_type=_F32,
                    precision=lax.Precision.HIGHEST)

        @pl.when(i == 0)
        def _():
            s_acc[...] = jnp.zeros_like(s_acc)
            q_acc[...] = jnp.zeros_like(q_acc)

        s_acc[...] += s
        q_acc[...] += q

    @pl.when(ph == 1)
    def _():
        mu = s_acc[...] / _EL
        var = q_acc[...] / _EL - mu * mu
        sc = g_ref[...] * lax.rsqrt(var + 1e-5)
        w = w_ref[...]
        wtop = w[0] * sc[0][:, None]
        wbot = w[1] * sc[1][:, None]
        p_ref[...] = jnp.dot(z, wtop, preferred_element_type=_F32,
                             precision=lax.Precision.HIGHEST)
        qo_ref[...] = jnp.dot(z, wbot, preferred_element_type=_F32,
                              precision=lax.Precision.HIGHEST)


def _tc_pq(z, counts, g1, w1):
    return pl.pallas_call(
        _tc_pq_body,
        grid=(2, _N // _BN),
        in_specs=[
            pl.BlockSpec((_BN, _D), lambda p, i: (i, 0)),
            pl.BlockSpec((_BN, 6), lambda p, i: (i, 0)),
            pl.BlockSpec((2, _D), lambda p, i: (0, 0)),
            pl.BlockSpec((2, _D, 32), lambda p, i: (0, 0, 0)),
        ],
        out_specs=(pl.BlockSpec((_BN, 32), lambda p, i: (i, 0)),
                   pl.BlockSpec((_BN, 32), lambda p, i: (i, 0))),
        out_shape=(jax.ShapeDtypeStruct((_N, 32), _F32),
                   jax.ShapeDtypeStruct((_N, 32), _F32)),
        scratch_shapes=[pltpu.VMEM((2, _D), _F32),
                        pltpu.VMEM((2, _D), _F32)],
    )(z, counts, g1, w1)


_BE = 2000


def _tc_final_body(r_ref, st_ref, g_ref, be_ref, w_ref, bb_ref, o_ref):
    st = st_ref[...][:, 0, :]
    ssum = jnp.sum(st[:, :32], axis=0, keepdims=True)
    qsum = jnp.sum(st[:, 32:], axis=0, keepdims=True)
    mu = ssum / _EL
    var = qsum / _EL - mu * mu
    sc = g_ref[...] * lax.rsqrt(var + 1e-5)
    sh = be_ref[...] - mu * sc
    c = jnp.maximum(r_ref[...] * sc + sh, 0.0)
    o_ref[...] = jnp.dot(c, w_ref[...], preferred_element_type=_F32, precision=lax.Precision.HIGHEST) + bb_ref[...]


def _tc_final(r, stats, g2, be2, w2, b2):
    return pl.pallas_call(
        _tc_final_body,
        grid=(_EL // _BE,),
        in_specs=[
            pl.BlockSpec((_BE, 32), lambda i: (i, 0)),
            pl.BlockSpec((_NW, 8, 64), lambda i: (0, 0, 0)),
            pl.BlockSpec((1, 32), lambda i: (0, 0)),
            pl.BlockSpec((1, 32), lambda i: (0, 0)),
            pl.BlockSpec((32, 2), lambda i: (0, 0)),
            pl.BlockSpec((1, 2), lambda i: (0, 0)),
        ],
        out_specs=pl.BlockSpec((_BE, 2), lambda i: (i, 0)),
        out_shape=jax.ShapeDtypeStruct((_EL, 2), _F32),
    )(r, stats, g2, be2, w2, b2)


def kernel(node_features, edge_index, edge_type, edge_label_index,
           ln_gamma, ln_beta, W_in, filt, W_layer,
           bn1_gamma, bn1_beta, W1, b1, bn2_gamma, bn2_beta, W2, b2):
    src = edge_index[0].astype(jnp.int32)
    dst = edge_index[1].astype(jnp.int32)
    et = edge_type.astype(jnp.int32)
    srcl = edge_label_index[0].astype(jnp.int32)
    dstl = edge_label_index[1].astype(jnp.int32)

    gidx = et * _N + src
    zeros_l = jnp.zeros((_CH, _D), _F32)

    eidx = jnp.arange(_E, dtype=jnp.int32)
    dump_n = _N + (eidx % 128)
    didx6 = jnp.stack(
        [jnp.where(et == t, dst, dump_n) for t in range(_T)] + [srcl, dstl])
    didx6 = jnp.transpose(didx6.reshape(_NSEG, _NW, _C_CHUNKS, _CH),
                          (1, 2, 0, 3)).reshape(-1)
    counts = _tc_count_reduce(_sc_count(didx6))
    counts = counts.reshape(_NSEG, _CNT_B).T

    dump_h = _HALF + (eidx % 48)
    didx2 = jnp.concatenate([
        jnp.where((dst >= c * _HALF) & (dst < (c + 1) * _HALF),
                  dst - c * _HALF, dump_h)
        for c in range(_NC)])
    g3 = jnp.broadcast_to(gidx.reshape(1, _NS, _EC_CHUNKS, _CH),
                          (_NC, _NS, _EC_CHUNKS, _CH))
    d3 = didx2.reshape(_NC, _NS, _EC_CHUNKS, _CH)
    comb = jnp.stack([g3, d3], axis=3).reshape(-1)
    lab = jnp.stack([srcl.reshape(_NW, _L_CHUNKS, _CH),
                     dstl.reshape(_NW, _L_CHUNKS, _CH)], axis=2).reshape(-1)

    h0 = _tc_prologue(node_features, ln_gamma.reshape(1, _D),
                      ln_beta.reshape(1, _D), W_in, filt)

    agg0 = _sc_layer(h0.reshape(_T * _N, _D), comb, zeros_l)
    h1 = _tc_layer(agg0, counts, filt, W_layer[0], l=0, last=False)
    agg1 = _sc_layer(h1.reshape(_T * _N, _D), comb, zeros_l)
    z = _tc_layer(agg1, counts, filt, W_layer[1], l=1, last=True)

    p_rows, q_rows = _tc_pq(z, counts, bn1_gamma.reshape(2, _D),
                            W1.reshape(2, _D, 32))
    r, stats = _sc_decode(p_rows, q_rows, lab)
    return _tc_final(r, stats, bn2_gamma.reshape(1, 32),
                     bn2_beta.reshape(1, 32), W2, b2.reshape(1, 2))


# speedup vs baseline: 4.9416x; 1.0400x over previous
import functools

import jax
import jax.numpy as jnp
from jax import lax
from jax.experimental import pallas as pl
from jax.experimental.pallas import tpu as pltpu
from jax.experimental.pallas import tpu_sc as plsc

_N = 10000
_D = 128
_T = 4
_E = 320000
_EL = 320000
_NC = 2
_NS = 16
_NW = _NC * _NS
_CH = 80
_F32 = jnp.float32

_NSEG = 6
_HALF = _N // _NC
_HROWS = _HALF + 48
_EC_PT = _E // _NS
_EC_CHUNKS = _EC_PT // _CH


def _sc_mesh():
    return plsc.VectorSubcoreMesh(core_axis_name="c", subcore_axis_name="s")


def _tile_chunks(sid, n_chunks, body):
    def jb(j, c0):
        c = sid + _NS * j
        @pl.when(c < n_chunks)
        def _():
            body(c)
        return c0
    lax.fori_loop(0, (n_chunks + _NS - 1) // _NS, jb, 0)


_C_PW = _E // _NW
_C_CHUNKS = _C_PW // _CH
_CNT_B = 10240


def _sc_count_body(didx6_hbm, out_hbm, *bufs):
    accs = bufs[0:_NSEG]
    idx_v = bufs[_NSEG:_NSEG + 5]
    isem = bufs[_NSEG + 5:_NSEG + 10]
    cid = lax.axis_index("c")
    sid = lax.axis_index("s")
    wid = sid * _NC + cid

    z16 = jnp.zeros((16,), _F32)
    for k in range(_NSEG):
        def zb(q, cc):
            accs[k][lax.shift_right_logical(q, 3),
                    pl.ds(lax.bitwise_and(q, 7) * 16, 16)] = z16
            return cc
        lax.fori_loop(0, 640, zb, 0)

    base_w = wid * _C_CHUNKS * (_NSEG * _CH)

    def blk(g, cc):
        c0 = g * 5
        idesc = []
        for b in range(5):
            idesc.append(pltpu.async_copy(
                didx6_hbm.at[pl.ds(base_w + (c0 + b) * (_NSEG * _CH),
                                   _NSEG * _CH)],
                idx_v[b], isem[b]))
        for b in range(5):
            idesc[b].wait()
            for k in range(_NSEG):
                for j in range(_CH // 16):
                    v = idx_v[b][pl.ds(k * _CH + j * 16, 16)]
                    cnts, last = plsc.scan_count(v)
                    plsc.addupdate_scatter(
                        accs[k],
                        [lax.shift_right_logical(v, 7),
                         lax.bitwise_and(v, 127)],
                        cnts.astype(_F32), mask=last)
        return cc
    lax.fori_loop(0, _C_CHUNKS // 5, blk, 0)

    for k in range(_NSEG):
        pltpu.sync_copy(accs[k], out_hbm.at[k * _NW + wid])


def _sc_count(didx6):
    scratch = ([pltpu.VMEM((80, 128), _F32) for _ in range(_NSEG)]
               + [pltpu.VMEM((_NSEG * _CH,), jnp.int32) for _ in range(5)]
               + [pltpu.SemaphoreType.DMA for _ in range(5)])
    return pl.kernel(
        _sc_count_body,
        out_type=jax.ShapeDtypeStruct((_NSEG * _NW, 80, 128), _F32),
        mesh=_sc_mesh(),
        scratch_types=scratch,
        compiler_params=pltpu.CompilerParams(needs_layout_passes=False),
    )(didx6)


def _tc_count_reduce_body(h_ref, o_ref):
    o_ref[...] = jnp.sum(h_ref[...], axis=0)[None]


def _tc_count_reduce(hists):
    return pl.pallas_call(
        _tc_count_reduce_body,
        grid=(_NSEG, 10),
        in_specs=[pl.BlockSpec((_NW, 8, 128), lambda k, i: (k, i, 0))],
        out_specs=pl.BlockSpec((1, 8, 128), lambda k, i: (k, i, 0)),
        out_shape=jax.ShapeDtypeStruct((_NSEG, 80, 128), _F32),
    )(hists)


_NB = 5


def _sc_layer_body(h4_hbm, comb_hbm, zeros_hbm, out_hbm, agg_sh, *bufs):
    idx_v = bufs[0:_NB]
    gidx_v = bufs[_NB:2 * _NB]
    didx_v = bufs[2 * _NB:3 * _NB]
    rows_v = bufs[3 * _NB:4 * _NB]
    zbuf_v = rows_v[0]
    isem = bufs[4 * _NB:4 * _NB + _NB]
    gsem = bufs[4 * _NB + _NB:4 * _NB + 2 * _NB]
    ssem = bufs[4 * _NB + 2 * _NB:4 * _NB + 3 * _NB]
    cid = lax.axis_index("c")
    sid = lax.axis_index("s")

    pltpu.sync_copy(zeros_hbm, zbuf_v)
    def zb(c):
        pltpu.sync_copy(zbuf_v, agg_sh.at[pl.ds(c * _CH, _CH)])
    _tile_chunks(sid, 62, zb)
    @pl.when(sid == 0)
    def _():
        pltpu.sync_copy(zeros_hbm.at[pl.ds(0, 40)],
                        agg_sh.at[pl.ds(4960, 40)])
        pltpu.sync_copy(zeros_hbm.at[pl.ds(0, 48)],
                        agg_sh.at[pl.ds(_HALF, 48)])
    plsc.subcore_barrier()

    base = (cid * _NS + sid) * _EC_CHUNKS * (2 * _CH)

    def blk(g, cc):
        c0 = g * _NB
        idesc = []
        for b in range(_NB):
            idesc.append(pltpu.async_copy(
                comb_hbm.at[pl.ds(base + (c0 + b) * (2 * _CH), 2 * _CH)],
                idx_v[b], isem[b]))
        gdesc = []
        for b in range(_NB):
            @pl.when(g > 0)
            def _(b=b):
                pltpu.make_async_copy(
                    rows_v[b], agg_sh.at[didx_v[b]], ssem[b]).wait()
            idesc[b].wait()
            for j in range(_CH // 16):
                gidx_v[b][pl.ds(j * 16, 16)] = idx_v[b][pl.ds(j * 16, 16)]
                didx_v[b][pl.ds(j * 16, 16)] = \
                    idx_v[b][pl.ds(_CH + j * 16, 16)]
            gdesc.append(pltpu.async_copy(
                h4_hbm.at[gidx_v[b]], rows_v[b], gsem[b]))
        for b in range(_NB):
            gdesc[b].wait()
            pltpu.async_copy(rows_v[b], agg_sh.at[didx_v[b]], ssem[b],
                             add=True)
        return cc
    lax.fori_loop(0, _EC_CHUNKS // _NB, blk, 0)
    for b in range(_NB):
        pltpu.make_async_copy(rows_v[b], agg_sh.at[didx_v[b]], ssem[b]).wait()
    plsc.subcore_barrier()

    def db(c):
        pltpu.sync_copy(agg_sh.at[pl.ds(c * _CH, _CH)], zbuf_v)
        pltpu.sync_copy(zbuf_v, out_hbm.at[pl.ds(cid * _HALF + c * _CH, _CH)])
    _tile_chunks(sid, 62, db)
    @pl.when(sid == 0)
    def _():
        pltpu.sync_copy(agg_sh.at[pl.ds(4960, 40)],
                        zbuf_v.at[pl.ds(0, 40)])
        pltpu.sync_copy(zbuf_v.at[pl.ds(0, 40)],
                        out_hbm.at[pl.ds(cid * _HALF + 4960, 40)])


def _sc_layer(h4, comb, zeros_l):
    scratch = ([pltpu.VMEM_SHARED((_HROWS, _D), _F32)]
               + [pltpu.VMEM((2 * _CH,), jnp.int32) for _ in range(_NB)]
               + [pltpu.VMEM((_CH,), jnp.int32) for _ in range(2 * _NB)]
               + [pltpu.VMEM((_CH, _D), _F32) for _ in range(_NB)]
               + [pltpu.SemaphoreType.DMA for _ in range(3 * _NB)])
    return pl.kernel(
        _sc_layer_body,
        out_type=jax.ShapeDtypeStruct((_N, _D), _F32),
        mesh=_sc_mesh(),
        scratch_types=scratch,
    )(h4, comb, zeros_l)


_L_PW = _EL // _NW
_NBD = 5
_L_CHUNKS = _L_PW // _CH


def _sc_decode_body(p_hbm, q_hbm, lab_hbm, r_hbm, stats_hbm,
                    p_sh, q_sh, stats_v, stage_v, *bufs):
    idx_v = bufs[0:_NBD]
    src_v = bufs[_NBD:2 * _NBD]
    dst_v = bufs[2 * _NBD:3 * _NBD]
    p_v = bufs[3 * _NBD:4 * _NBD]
    q_v = bufs[4 * _NBD:5 * _NBD]
    r_v = bufs[5 * _NBD:6 * _NBD]
    isem = bufs[6 * _NBD:7 * _NBD]
    psem = bufs[7 * _NBD:8 * _NBD]
    qsem = bufs[8 * _NBD:9 * _NBD]
    wsem = bufs[9 * _NBD:10 * _NBD]
    cid = lax.axis_index("c")
    sid = lax.axis_index("s")
    wid = sid * _NC + cid
    base_w = wid * _L_PW

    def stage(c):
        pltpu.sync_copy(p_hbm.at[pl.ds(c * 200, 200)], stage_v)
        pltpu.sync_copy(stage_v, p_sh.at[pl.ds(c * 200, 200)])
        pltpu.sync_copy(q_hbm.at[pl.ds(c * 200, 200)], stage_v)
        pltpu.sync_copy(stage_v, q_sh.at[pl.ds(c * 200, 200)])
    _tile_chunks(sid, _N // 200, stage)

    z16 = jnp.zeros((16,), _F32)
    for i in range(8):
        for j in range(4):
            stats_v[i, pl.ds(j * 16, 16)] = z16
    plsc.subcore_barrier()

    base_i = wid * _L_CHUNKS * (2 * _CH)

    def blk(g, carry):
        c0 = g * _NBD
        idesc = []
        for b in range(_NBD):
            idesc.append(pltpu.async_copy(
                lab_hbm.at[pl.ds(base_i + (c0 + b) * (2 * _CH), 2 * _CH)],
                idx_v[b], isem[b]))
        gdesc = []
        for b in range(_NBD):
            @pl.when(g > 0)
            def _(b=b):
                pltpu.make_async_copy(
                    r_v[b], r_hbm.at[pl.ds(base_w, _CH)], wsem[b]).wait()
            idesc[b].wait()
            for j in range(_CH // 16):
                src_v[b][pl.ds(j * 16, 16)] = idx_v[b][pl.ds(j * 16, 16)]
                dst_v[b][pl.ds(j * 16, 16)] = \
                    idx_v[b][pl.ds(_CH + j * 16, 16)]
            gdesc.append((pltpu.async_copy(p_sh.at[src_v[b]], p_v[b], psem[b]),
                          pltpu.async_copy(q_sh.at[dst_v[b]], q_v[b], qsem[b])))
        for b in range(_NBD):
            gdesc[b][0].wait()
            gdesc[b][1].wait()

            def eb(i, car):
                s0, s1, t0, t1 = car
                a0 = p_v[b][i, pl.ds(0, 16)] + q_v[b][i, pl.ds(0, 16)]
                a1 = p_v[b][i, pl.ds(16, 16)] + q_v[b][i, pl.ds(16, 16)]
                r_v[b][i, pl.ds(0, 16)] = a0
                r_v[b][i, pl.ds(16, 16)] = a1
                return (s0 + a0, s1 + a1, t0 + a0 * a0, t1 + a1 * a1)

            carry = lax.fori_loop(0, _CH, eb, carry)
            pltpu.async_copy(
                r_v[b], r_hbm.at[pl.ds(base_w + (c0 + b) * _CH, _CH)],
                wsem[b])
        return carry

    s0, s1, t0, t1 = lax.fori_loop(0, _L_CHUNKS // _NBD, blk,
                                   (z16, z16, z16, z16))
    for b in range(_NBD):
        pltpu.make_async_copy(
            r_v[b], r_hbm.at[pl.ds(base_w, _CH)], wsem[b]).wait()
    stats_v[0, pl.ds(0, 16)] = s0
    stats_v[0, pl.ds(16, 16)] = s1
    stats_v[0, pl.ds(32, 16)] = t0
    stats_v[0, pl.ds(48, 16)] = t1
    pltpu.sync_copy(stats_v, stats_hbm.at[wid])


def _sc_decode(p, q, lab):
    scratch = ([pltpu.VMEM_SHARED((_N, 32), _F32),
                pltpu.VMEM_SHARED((_N, 32), _F32),
                pltpu.VMEM((8, 64), _F32),
                pltpu.VMEM((200, 32), _F32)]
               + [pltpu.VMEM((2 * _CH,), jnp.int32) for _ in range(_NBD)]
               + [pltpu.VMEM((_CH,), jnp.int32) for _ in range(2 * _NBD)]
               + [pltpu.VMEM((_CH, 32), _F32) for _ in range(3 * _NBD)]
               + [pltpu.SemaphoreType.DMA for _ in range(4 * _NBD)])
    return pl.kernel(
        _sc_decode_body,
        out_type=(jax.ShapeDtypeStruct((_EL, 32), _F32),
                  jax.ShapeDtypeStruct((_NW, 8, 64), _F32)),
        mesh=_sc_mesh(),
        scratch_types=scratch,
        compiler_params=pltpu.CompilerParams(use_tc_tiling_on_sc=False),
    )(p, q, lab)


_BN = 1000


def _softmax_rows(f):
    m = jnp.max(f, axis=-1, keepdims=True)
    e = jnp.exp(f - m)
    return e / jnp.sum(e, axis=-1, keepdims=True)


def _tc_prologue_body(nf_ref, g_ref, b_ref, w_ref, f_ref, out_ref):
    x = nf_ref[...]
    mu = jnp.mean(x, axis=-1, keepdims=True)
    var = jnp.mean((x - mu) * (x - mu), axis=-1, keepdims=True)
    x = (x - mu) * lax.rsqrt(var + 1e-5) * g_ref[...] + b_ref[...]
    h = jnp.dot(x, w_ref[...], preferred_element_type=_F32, precision=lax.Precision.HIGHEST)
    a = _softmax_rows(f_ref[...])[0]
    out_ref[...] = a[:, None, None] * h[None, :, :]


def _tc_prologue(nf, ln_g, ln_b, w_in, filt):
    return pl.pallas_call(
        _tc_prologue_body,
        grid=(_N // _BN,),
        in_specs=[
            pl.BlockSpec((_BN, _D), lambda i: (i, 0)),
            pl.BlockSpec((1, _D), lambda i: (0, 0)),
            pl.BlockSpec((1, _D), lambda i: (0, 0)),
            pl.BlockSpec((_D, _D), lambda i: (0, 0)),
            pl.BlockSpec((2, _T), lambda i: (0, 0)),
        ],
        out_specs=pl.BlockSpec((_T, _BN, _D), lambda i: (0, i, 0)),
        out_shape=jax.ShapeDtypeStruct((_T, _N, _D), _F32),
    )(nf, ln_g, ln_b, w_in, filt)


def _tc_layer_body(l, last, agg_ref, cnt_ref, f_ref, w_ref, out_ref):
    agg = agg_ref[...]
    a_all = _softmax_rows(f_ref[...])
    ac = a_all[l]
    cnt = cnt_ref[...]
    deg = jnp.sum(ac[None, :] * cnt[:, :_T], axis=1)
    agg = agg / jnp.clip(deg, 1e-6, None)[:, None]
    h = jnp.maximum(jnp.dot(agg, w_ref[...], preferred_element_type=_F32, precision=lax.Precision.HIGHEST), 0.0)
    if last:
        out_ref[...] = h
    else:
        an = a_all[l + 1]
        out_ref[...] = an[:, None, None] * h[None, :, :]


def _tc_layer(agg, counts, filt, w_l, l, last):
    if last:
        out_spec = pl.BlockSpec((_BN, _D), lambda i: (i, 0))
        out_shape = jax.ShapeDtypeStruct((_N, _D), _F32)
    else:
        out_spec = pl.BlockSpec((_T, _BN, _D), lambda i: (0, i, 0))
        out_shape = jax.ShapeDtypeStruct((_T, _N, _D), _F32)
    return pl.pallas_call(
        functools.partial(_tc_layer_body, l, last),
        grid=(_N // _BN,),
        in_specs=[
            pl.BlockSpec((_BN, _D), lambda i: (i, 0)),
            pl.BlockSpec((_BN, 6), lambda i: (i, 0)),
            pl.BlockSpec((2, _T), lambda i: (0, 0)),
            pl.BlockSpec((_D, _D), lambda i: (0, 0)),
        ],
        out_specs=out_spec,
        out_shape=out_shape,
    )(agg, counts, filt, w_l)


def _tc_pq_body(z_ref, c_ref, g_ref, w_ref, p_ref, qo_ref, s_acc, q_acc):
    ph = pl.program_id(0)
    i = pl.program_id(1)
    z = z_ref[...]

    @pl.when(ph == 0)
    def _():
        cnt = c_ref[...][:, 4:6].T
        s = jnp.dot(cnt, z, preferred_element_type=_F32,
                    precision=lax.Precision.HIGHEST)
        q = jnp.dot(cnt, z * z, preferred_element_type=_F32,
                    precision=lax.Precision.HIGHEST)

        @pl.when(i == 0)
        def _():
            s_acc[...] = jnp.zeros_like(s_acc)
            q_acc[...] = jnp.zeros_like(q_acc)

        s_acc[...] += s
        q_acc[...] += q

    @pl.when(ph == 1)
    def _():
        mu = s_acc[...] / _EL
        var = q_acc[...] / _EL - mu * mu
        sc = g_ref[...] * lax.rsqrt(var + 1e-5)
        w = w_ref[...]
        wtop = w[0] * sc[0][:, None]
        wbot = w[1] * sc[1][:, None]
        p_ref[...] = jnp.dot(z, wtop, preferred_element_type=_F32,
                             precision=lax.Precision.HIGHEST)
        qo_ref[...] = jnp.dot(z, wbot, preferred_element_type=_F32,
                              precision=lax.Precision.HIGHEST)


def _tc_pq(z, counts, g1, w1):
    return pl.pallas_call(
        _tc_pq_body,
        grid=(2, _N // _BN),
        in_specs=[
            pl.BlockSpec((_BN, _D), lambda p, i: (i, 0)),
            pl.BlockSpec((_BN, 6), lambda p, i: (i, 0)),
            pl.BlockSpec((2, _D), lambda p, i: (0, 0)),
            pl.BlockSpec((2, _D, 32), lambda p, i: (0, 0, 0)),
        ],
        out_specs=(pl.BlockSpec((_BN, 32), lambda p, i: (i, 0)),
                   pl.BlockSpec((_BN, 32), lambda p, i: (i, 0))),
        out_shape=(jax.ShapeDtypeStruct((_N, 32), _F32),
                   jax.ShapeDtypeStruct((_N, 32), _F32)),
        scratch_shapes=[pltpu.VMEM((2, _D), _F32),
                        pltpu.VMEM((2, _D), _F32)],
    )(z, counts, g1, w1)


_BE = 2000


def _tc_final_body(r_ref, st_ref, g_ref, be_ref, w_ref, bb_ref, o_ref):
    st = st_ref[...][:, 0, :]
    ssum = jnp.sum(st[:, :32], axis=0, keepdims=True)
    qsum = jnp.sum(st[:, 32:], axis=0, keepdims=True)
    mu = ssum / _EL
    var = qsum / _EL - mu * mu
    sc = g_ref[...] * lax.rsqrt(var + 1e-5)
    sh = be_ref[...] - mu * sc
    c = jnp.maximum(r_ref[...] * sc + sh, 0.0)
    o_ref[...] = jnp.dot(c, w_ref[...], preferred_element_type=_F32, precision=lax.Precision.HIGHEST) + bb_ref[...]


def _tc_final(r, stats, g2, be2, w2, b2):
    return pl.pallas_call(
        _tc_final_body,
        grid=(_EL // _BE,),
        in_specs=[
            pl.BlockSpec((_BE, 32), lambda i: (i, 0)),
            pl.BlockSpec((_NW, 8, 64), lambda i: (0, 0, 0)),
            pl.BlockSpec((1, 32), lambda i: (0, 0)),
            pl.BlockSpec((1, 32), lambda i: (0, 0)),
            pl.BlockSpec((32, 2), lambda i: (0, 0)),
            pl.BlockSpec((1, 2), lambda i: (0, 0)),
        ],
        out_specs=pl.BlockSpec((_BE, 2), lambda i: (i, 0)),
        out_shape=jax.ShapeDtypeStruct((_EL, 2), _F32),
    )(r, stats, g2, be2, w2, b2)


def kernel(node_features, edge_index, edge_type, edge_label_index,
           ln_gamma, ln_beta, W_in, filt, W_layer,
           bn1_gamma, bn1_beta, W1, b1, bn2_gamma, bn2_beta, W2, b2):
    src = edge_index[0].astype(jnp.int32)
    dst = edge_index[1].astype(jnp.int32)
    et = edge_type.astype(jnp.int32)
    srcl = edge_label_index[0].astype(jnp.int32)
    dstl = edge_label_index[1].astype(jnp.int32)

    gidx = et * _N + src
    zeros_l = jnp.zeros((_CH, _D), _F32)

    eidx = jnp.arange(_E, dtype=jnp.int32)
    dump_n = _N + (eidx % 128)
    didx6 = jnp.stack(
        [jnp.where(et == t, dst, dump_n) for t in range(_T)] + [srcl, dstl])
    didx6 = jnp.transpose(didx6.reshape(_NSEG, _NW, _C_CHUNKS, _CH),
                          (1, 2, 0, 3)).reshape(-1)
    counts = _tc_count_reduce(_sc_count(didx6))
    counts = counts.reshape(_NSEG, _CNT_B).T

    dump_h = _HALF + (eidx % 48)
    didx2 = jnp.concatenate([
        jnp.where((dst >= c * _HALF) & (dst < (c + 1) * _HALF),
                  dst - c * _HALF, dump_h)
        for c in range(_NC)])
    g3 = jnp.broadcast_to(gidx.reshape(1, _NS, _EC_CHUNKS, _CH),
                          (_NC, _NS, _EC_CHUNKS, _CH))
    d3 = didx2.reshape(_NC, _NS, _EC_CHUNKS, _CH)
    comb = jnp.stack([g3, d3], axis=3).reshape(-1)
    lab = jnp.stack([srcl.reshape(_NW, _L_CHUNKS, _CH),
                     dstl.reshape(_NW, _L_CHUNKS, _CH)], axis=2).reshape(-1)

    h0 = _tc_prologue(node_features, ln_gamma.reshape(1, _D),
                      ln_beta.reshape(1, _D), W_in, filt)

    agg0 = _sc_layer(h0.reshape(_T * _N, _D), comb, zeros_l)
    h1 = _tc_layer(agg0, counts, filt, W_layer[0], l=0, last=False)
    agg1 = _sc_layer(h1.reshape(_T * _N, _D), comb, zeros_l)
    z = _tc_layer(agg1, counts, filt, W_layer[1], l=1, last=True)

    p_rows, q_rows = _tc_pq(z, counts, bn1_gamma.reshape(2, _D),
                            W1.reshape(2, _D, 32))
    r, stats = _sc_decode(p_rows, q_rows, lab)
    return _tc_final(r, stats, bn2_gamma.reshape(1, 32),
                     bn2_beta.reshape(1, 32), W2, b2.reshape(1, 2))
